# Initial kernel scaffold; baseline (speedup 1.0000x reference)
import jax
import jax.numpy as jnp
from jax.experimental import pallas as pl


def kernel(x, edge_index, W1, att_src1, att_dst1, bias1, W2, att_src2, att_dst2, bias2):
    raise NotImplementedError("write your pallas kernel here")


# speedup vs baseline: 26.3284x; 26.3284x over previous
import dataclasses
import functools

import jax
import jax.numpy as jnp
from jax import lax
from jax.experimental import pallas as pl
from jax.experimental.pallas import tpu as pltpu
from jax.experimental.pallas import tpu_sc as plsc

N = 10000
NP = 10048
DUMMY = 10000
E = 320000
NTILES = 32
NSUB = 16
BE = 128
BLOCKS_PER_TILE = 81
EPT = BE * BLOCKS_PER_TILE
EP = EPT * NTILES
ROWS_PER_SUB = NP // NSUB

RW1, H1, C1 = 80, 8, 8
RW2, H2, C2 = 144, 1, 128
RB = 64
_HP = lax.Precision.HIGHEST


def _dense1_body(x_ref, w_ref, asv_ref, adv_ref, xls_ref, adst_ref):
    xl = jnp.dot(x_ref[...], w_ref[...], precision=_HP)
    asr = jnp.dot(xl, asv_ref[...], precision=_HP)
    adr = jnp.dot(xl, adv_ref[...], precision=_HP)
    xls_ref[...] = jnp.concatenate(
        [xl, asr, jnp.zeros((RB, 8), jnp.float32)], axis=1)
    adst_ref[...] = adr


def _dense1(xp, W1, asv1, adv1p):
    return pl.pallas_call(
        _dense1_body,
        grid=(NP // RB,),
        in_specs=[
            pl.BlockSpec((RB, 128), lambda i: (i, 0)),
            pl.BlockSpec((128, 64), lambda i: (0, 0)),
            pl.BlockSpec((64, 8), lambda i: (0, 0)),
            pl.BlockSpec((64, 16), lambda i: (0, 0)),
        ],
        out_specs=[
            pl.BlockSpec((RB, RW1), lambda i: (i, 0)),
            pl.BlockSpec((RB, 16), lambda i: (i, 0)),
        ],
        out_shape=[
            jax.ShapeDtypeStruct((NP, RW1), jnp.float32),
            jax.ShapeDtypeStruct((NP, 16), jnp.float32),
        ],
    )(xp, W1, asv1, adv1p)


def _mid_body(pa_ref, pb_ref, b1_ref, w2_ref, r8_ref, asv2_ref, adv2_ref,
              xls2_ref, adst2_ref):
    acc = pa_ref[...] + pb_ref[...]
    numer = acc[:, 0:64]
    den8 = acc[:, 64:72]
    db = jnp.dot(den8, r8_ref[...], precision=_HP)
    hpre = numer / (db + 1e-16) + b1_ref[...]
    h = jnp.where(hpre > 0, hpre, jnp.exp(jnp.minimum(hpre, 0.0)) - 1.0)
    xl2 = jnp.dot(h, w2_ref[...], precision=_HP)
    asr2 = jnp.dot(xl2, asv2_ref[...], precision=_HP)
    adr2 = jnp.dot(xl2, adv2_ref[...], precision=_HP)
    xls2_ref[...] = jnp.concatenate([xl2, asr2], axis=1)
    adst2_ref[...] = adr2


def _mid(pa, pb, b1, W2, r8, asv2p, adv2p):
    return pl.pallas_call(
        _mid_body,
        grid=(NP // RB,),
        in_specs=[
            pl.BlockSpec((RB, RW1), lambda i: (i, 0)),
            pl.BlockSpec((RB, RW1), lambda i: (i, 0)),
            pl.BlockSpec((1, 64), lambda i: (0, 0)),
            pl.BlockSpec((64, 128), lambda i: (0, 0)),
            pl.BlockSpec((8, 64), lambda i: (0, 0)),
            pl.BlockSpec((128, 16), lambda i: (0, 0)),
            pl.BlockSpec((128, 16), lambda i: (0, 0)),
        ],
        out_specs=[
            pl.BlockSpec((RB, RW2), lambda i: (i, 0)),
            pl.BlockSpec((RB, 16), lambda i: (i, 0)),
        ],
        out_shape=[
            jax.ShapeDtypeStruct((NP, RW2), jnp.float32),
            jax.ShapeDtypeStruct((NP, 16), jnp.float32),
        ],
    )(pa, pb, b1, W2, r8, asv2p, adv2p)


def _fin_body(pa_ref, pb_ref, b2_ref, out_ref):
    acc = pa_ref[...] + pb_ref[...]
    numer = acc[:, 0:128]
    den = acc[:, 128:129]
    out_ref[...] = numer / (den + 1e-16) + b2_ref[...]


def _fin(pa, pb, b2):
    return pl.pallas_call(
        _fin_body,
        grid=(NP // RB,),
        in_specs=[
            pl.BlockSpec((RB, RW2), lambda i: (i, 0)),
            pl.BlockSpec((RB, RW2), lambda i: (i, 0)),
            pl.BlockSpec((1, 128), lambda i: (0, 0)),
        ],
        out_specs=pl.BlockSpec((RB, 128), lambda i: (i, 0)),
        out_shape=jax.ShapeDtypeStruct((NP, 128), jnp.float32),
    )(pa, pb, b2)


def _edge_body(xls_hbm, adst_hbm, src_hbm, dst_hbm, zer_hbm, out_hbm,
               srcv, dstv, xg, ag, msg, wbuf, acc, sem1, sem2,
               *, rw, heads, acol):
    cid = lax.axis_index("c")
    sid = lax.axis_index("s")
    wid = sid * 2 + cid

    pltpu.sync_copy(zer_hbm.at[pl.ds(sid * ROWS_PER_SUB, ROWS_PER_SUB)],
                    acc.at[pl.ds(sid * ROWS_PER_SUB, ROWS_PER_SUB)])
    plsc.subcore_barrier()

    lane = lax.iota(jnp.int32, 16)
    nch = acol // 16
    if heads == 8:
        selpats = [(lane >> 3) + 2 * k for k in range(nch)]
        tailpat = lane & 7
    else:
        selpats = None
        zpat = lane * 0

    ebase = wid * EPT

    @pl.loop(0, BLOCKS_PER_TILE)
    def _block(b):
        base = ebase + b * BE
        pltpu.sync_copy(src_hbm.at[pl.ds(base, BE)], srcv)
        pltpu.sync_copy(dst_hbm.at[pl.ds(base, BE)], dstv)
        cp1 = pltpu.async_copy(xls_hbm.at[srcv], xg, sem1)
        cp2 = pltpu.async_copy(adst_hbm.at[dstv], ag, sem2)
        cp1.wait()
        cp2.wait()
        for g in range(BE // 16):
            eidx = lane + g * 16
            for h in range(heads):
                a_s = plsc.load_gather(
                    xg, [eidx, jnp.full((16,), acol + h, jnp.int32)])
                a_d = plsc.load_gather(
                    ag, [eidx, jnp.full((16,), h, jnp.int32)])
                s = a_s + a_d
                w = jnp.exp(jnp.maximum(s, 0.2 * s))
                wbuf[h, pl.ds(g * 16, 16)] = w
        for e in range(BE):
            esp = jnp.full((16,), e, jnp.int32)
            if heads == 8:
                for k in range(nch):
                    wsel = plsc.load_gather(wbuf, [selpats[k], esp])
                    msg[e, pl.ds(k * 16, 16)] = xg[e, pl.ds(k * 16, 16)] * wsel
                msg[e, pl.ds(acol, 16)] = plsc.load_gather(wbuf, [tailpat, esp])
            else:
                wsel = plsc.load_gather(wbuf, [zpat, esp])
                for k in range(nch):
                    msg[e, pl.ds(k * 16, 16)] = xg[e, pl.ds(k * 16, 16)] * wsel
                msg[e, pl.ds(acol, 16)] = wsel
        pltpu.sync_copy(msg, acc.at[dstv], add=True)

    plsc.subcore_barrier()
    pltpu.sync_copy(
        acc.at[pl.ds(sid * ROWS_PER_SUB, ROWS_PER_SUB)],
        out_hbm.at[pl.ds(cid * NP + sid * ROWS_PER_SUB, ROWS_PER_SUB)])


def _edge_pass(xls, adst, srcp, dstp, zer, rw, heads, acol):
    mesh = plsc.VectorSubcoreMesh(core_axis_name="c", subcore_axis_name="s")
    cp = pltpu.CompilerParams()
    if "needs_layout_passes" in pltpu.CompilerParams.__dataclass_fields__:
        cp = dataclasses.replace(cp, needs_layout_passes=False)
    if "use_tc_tiling_on_sc" in pltpu.CompilerParams.__dataclass_fields__:
        cp = dataclasses.replace(cp, use_tc_tiling_on_sc=False)
    kern = pl.kernel(
        functools.partial(_edge_body, rw=rw, heads=heads, acol=acol),
        out_type=jax.ShapeDtypeStruct((2 * NP, rw), jnp.float32),
        mesh=mesh,
        scratch_types=[
            pltpu.VMEM((BE,), jnp.int32),
            pltpu.VMEM((BE,), jnp.int32),
            pltpu.VMEM((BE, rw), jnp.float32),
            pltpu.VMEM((BE, 16), jnp.float32),
            pltpu.VMEM((BE, rw), jnp.float32),
            pltpu.VMEM((8, BE), jnp.float32),
            pltpu.VMEM_SHARED((NP, rw), jnp.float32),
            pltpu.SemaphoreType.DMA,
            pltpu.SemaphoreType.DMA,
        ],
        compiler_params=cp,
    )
    return kern(xls, adst, srcp, dstp, zer)


def kernel(x, edge_index, W1, att_src1, att_dst1, bias1,
           W2, att_src2, att_dst2, bias2):
    f32 = jnp.float32
    xp = jnp.pad(x, ((0, NP - N), (0, 0)))
    eye8 = jnp.eye(8, dtype=f32)
    asv1 = (att_src1[0][:, :, None] * eye8[:, None, :]).reshape(64, 8)
    adv1 = (att_dst1[0][:, :, None] * eye8[:, None, :]).reshape(64, 8)
    adv1p = jnp.pad(adv1, ((0, 0), (0, 8)))
    r8 = jnp.repeat(eye8, 8, axis=1)
    asv2p = jnp.pad(att_src2[0].reshape(128, 1), ((0, 0), (0, 15)))
    adv2p = jnp.pad(att_dst2[0].reshape(128, 1), ((0, 0), (0, 15)))
    loop = jnp.arange(N, dtype=jnp.int32)
    ei = edge_index.astype(jnp.int32)
    pad = jnp.full((EP - E - N,), DUMMY, jnp.int32)
    srcp = jnp.concatenate([ei[0], loop, pad])
    dstp = jnp.concatenate([ei[1], loop, pad])
    zer1 = jnp.zeros((NP, RW1), f32)
    zer2 = jnp.zeros((NP, RW2), f32)
    b1 = bias1.reshape(1, 64)
    b2 = bias2.reshape(1, 128)

    xls1, adst1 = _dense1(xp, W1, asv1, adv1p)
    p1 = _edge_pass(xls1, adst1, srcp, dstp, zer1, RW1, H1, H1 * C1)
    xls2, adst2 = _mid(p1[:NP], p1[NP:], b1, W2, r8, asv2p, adv2p)
    p2 = _edge_pass(xls2, adst2, srcp, dstp, zer2, RW2, H2, H2 * C2)
    out = _fin(p2[:NP], p2[NP:], b2)
    return out[:N]


# speedup vs baseline: 28.2457x; 1.0728x over previous
import dataclasses
import functools

import jax
import jax.numpy as jnp
from jax import lax
from jax.experimental import pallas as pl
from jax.experimental.pallas import tpu as pltpu
from jax.experimental.pallas import tpu_sc as plsc

N = 10000
NP = 10048
DUMMY = 10000
E = 320000
NTILES = 32
NSUB = 16
EPT = 10496
EP = EPT * NTILES
BE1, NBLK1 = 128, 82
BE2, NBLK2 = 32, 328
ROWS_PER_SUB = NP // NSUB

RW1, H1, C1 = 80, 8, 8
RW2, H2, C2 = 144, 1, 128
RB = 64


def _dot3(a, b):
    bf, f = jnp.bfloat16, jnp.float32
    ah = a.astype(bf)
    ar = (a - ah.astype(f)).astype(bf)
    bh = b.astype(bf)
    br = (b - bh.astype(f)).astype(bf)
    return (jnp.dot(ah, bh, preferred_element_type=f)
            + jnp.dot(ah, br, preferred_element_type=f)
            + jnp.dot(ar, bh, preferred_element_type=f))


def _dense1_body(x_ref, w_ref, asv_ref, adv_ref, xls_ref, adst_ref):
    xl = _dot3(x_ref[...], w_ref[...])
    asr = _dot3(xl, asv_ref[...])
    adr = _dot3(xl, adv_ref[...])
    xls_ref[...] = jnp.concatenate(
        [xl, asr, jnp.zeros((RB, 8), jnp.float32)], axis=1)
    adst_ref[...] = adr


def _dense1(xp, W1, asv1, adv1p):
    return pl.pallas_call(
        _dense1_body,
        grid=(NP // RB,),
        in_specs=[
            pl.BlockSpec((RB, 128), lambda i: (i, 0)),
            pl.BlockSpec((128, 64), lambda i: (0, 0)),
            pl.BlockSpec((64, 8), lambda i: (0, 0)),
            pl.BlockSpec((64, 16), lambda i: (0, 0)),
        ],
        out_specs=[
            pl.BlockSpec((RB, RW1), lambda i: (i, 0)),
            pl.BlockSpec((RB, 16), lambda i: (i, 0)),
        ],
        out_shape=[
            jax.ShapeDtypeStruct((NP, RW1), jnp.float32),
            jax.ShapeDtypeStruct((NP, 16), jnp.float32),
        ],
    )(xp, W1, asv1, adv1p)


def _mid_body(pa_ref, pb_ref, b1_ref, w2_ref, r8_ref, asv2_ref, adv2_ref,
              xls2_ref, adst2_ref):
    acc = pa_ref[...] + pb_ref[...]
    numer = acc[:, 0:64]
    den8 = acc[:, 64:72]
    db = _dot3(den8, r8_ref[...])
    hpre = numer / (db + 1e-16) + b1_ref[...]
    h = jnp.where(hpre > 0, hpre, jnp.exp(jnp.minimum(hpre, 0.0)) - 1.0)
    xl2 = _dot3(h, w2_ref[...])
    asr2 = _dot3(xl2, asv2_ref[...])
    adr2 = _dot3(xl2, adv2_ref[...])
    xls2_ref[...] = jnp.concatenate([xl2, asr2], axis=1)
    adst2_ref[...] = adr2


def _mid(pa, pb, b1, W2, r8, asv2p, adv2p):
    return pl.pallas_call(
        _mid_body,
        grid=(NP // RB,),
        in_specs=[
            pl.BlockSpec((RB, RW1), lambda i: (i, 0)),
            pl.BlockSpec((RB, RW1), lambda i: (i, 0)),
            pl.BlockSpec((1, 64), lambda i: (0, 0)),
            pl.BlockSpec((64, 128), lambda i: (0, 0)),
            pl.BlockSpec((8, 64), lambda i: (0, 0)),
            pl.BlockSpec((128, 16), lambda i: (0, 0)),
            pl.BlockSpec((128, 16), lambda i: (0, 0)),
        ],
        out_specs=[
            pl.BlockSpec((RB, RW2), lambda i: (i, 0)),
            pl.BlockSpec((RB, 16), lambda i: (i, 0)),
        ],
        out_shape=[
            jax.ShapeDtypeStruct((NP, RW2), jnp.float32),
            jax.ShapeDtypeStruct((NP, 16), jnp.float32),
        ],
    )(pa, pb, b1, W2, r8, asv2p, adv2p)


def _fin_body(pa_ref, pb_ref, b2_ref, out_ref):
    acc = pa_ref[...] + pb_ref[...]
    numer = acc[:, 0:128]
    den = acc[:, 128:129]
    out_ref[...] = numer / (den + 1e-16) + b2_ref[...]


def _fin(pa, pb, b2):
    return pl.pallas_call(
        _fin_body,
        grid=(NP // RB,),
        in_specs=[
            pl.BlockSpec((RB, RW2), lambda i: (i, 0)),
            pl.BlockSpec((RB, RW2), lambda i: (i, 0)),
            pl.BlockSpec((1, 128), lambda i: (0, 0)),
        ],
        out_specs=pl.BlockSpec((RB, 128), lambda i: (i, 0)),
        out_shape=jax.ShapeDtypeStruct((NP, 128), jnp.float32),
    )(pa, pb, b2)


def _edge_body(xls_hbm, adst_hbm, e3_hbm, zer_hbm, out_hbm,
               idxv0, idxv1, dsts0, dsts1, xg0, xg1, ag0, ag1,
               msg0, msg1, wbuf, acc,
               semi0, semi1, semx0, semx1, sema0, sema1, semsc0, semsc1,
               *, rw, heads, acol, be, nblk):
    idxv = (idxv0, idxv1)
    dsts = (dsts0, dsts1)
    xg = (xg0, xg1)
    ag = (ag0, ag1)
    msg = (msg0, msg1)
    semi = (semi0, semi1)
    semx = (semx0, semx1)
    sema = (sema0, sema1)
    semsc = (semsc0, semsc1)

    cid = lax.axis_index("c")
    sid = lax.axis_index("s")
    wid = sid * 2 + cid

    pltpu.sync_copy(zer_hbm.at[pl.ds(sid * ROWS_PER_SUB, ROWS_PER_SUB)],
                    acc.at[pl.ds(sid * ROWS_PER_SUB, ROWS_PER_SUB)])
    plsc.subcore_barrier()

    lane = lax.iota(jnp.int32, 16)
    nch = acol // 16
    if heads == 8:
        selpats = [(lane >> 3) + 2 * k for k in range(nch)]
        tailpat = lane & 7

    bbase = wid * nblk
    NBLK = nblk

    def issue_idx(t, s):
        pltpu.async_copy(e3_hbm.at[bbase + t], idxv[s], semi[s])

    def wait_idx(s):
        pltpu.make_async_copy(e3_hbm.at[bbase], idxv[s], semi[s]).wait()

    def issue_gathers(s):
        pltpu.async_copy(xls_hbm.at[idxv[s].at[0]], xg[s], semx[s])
        pltpu.async_copy(adst_hbm.at[idxv[s].at[1]], ag[s], sema[s])

    def wait_gathers(s):
        pltpu.make_async_copy(xls_hbm.at[idxv[s].at[0]], xg[s], semx[s]).wait()
        pltpu.make_async_copy(adst_hbm.at[idxv[s].at[1]], ag[s], sema[s]).wait()

    def issue_scatter(s):
        pltpu.async_copy(msg[s], acc.at[dsts[s]], semsc[s], add=True)

    def wait_scatter(s):
        pltpu.make_async_copy(msg[s], acc.at[dsts[s]], semsc[s]).wait()

    issue_idx(0, 0)
    issue_idx(1, 1)
    wait_idx(0)
    issue_gathers(0)

    def halfbody(t, s):
        o = 1 - s

        @pl.when(t + 1 < NBLK)
        def _():
            wait_idx(o)
            issue_gathers(o)

        wait_gathers(s)
        for g in range(be // 16):
            dsts[s][pl.ds(g * 16, 16)] = idxv[s][1, pl.ds(g * 16, 16)]

        @pl.when(t + 2 < NBLK)
        def _():
            issue_idx(t + 2, s)

        for g in range(be // 16):
            eidx = lane + g * 16
            for h in range(heads):
                a_s = plsc.load_gather(
                    xg[s], [eidx, jnp.full((16,), acol + h, jnp.int32)])
                a_d = plsc.load_gather(
                    ag[s], [eidx, jnp.full((16,), h, jnp.int32)])
                sv = a_s + a_d
                w = jnp.exp(jnp.maximum(sv, 0.2 * sv))
                if heads == 8:
                    wbuf[h, pl.ds(g * 16, 16)] = w
                else:
                    for j in range(16):
                        wbuf[j, pl.ds(g * 16, 16)] = w
        @pl.when(t >= 2)
        def _():
            wait_scatter(s)

        for e in range(be):
            esp = jnp.full((16,), e, jnp.int32)
            if heads == 8:
                for k in range(nch):
                    wsel = plsc.load_gather(wbuf, [selpats[k], esp])
                    msg[s][e, pl.ds(k * 16, 16)] = (
                        xg[s][e, pl.ds(k * 16, 16)] * wsel)
                msg[s][e, pl.ds(acol, 16)] = plsc.load_gather(
                    wbuf, [tailpat, esp])
            else:
                wsel = plsc.load_gather(wbuf, [lane, esp])
                for k in range(nch):
                    msg[s][e, pl.ds(k * 16, 16)] = (
                        xg[s][e, pl.ds(k * 16, 16)] * wsel)
                msg[s][e, pl.ds(acol, 16)] = wsel

        issue_scatter(s)

    @pl.loop(0, NBLK // 2)
    def _pair(tt):
        halfbody(2 * tt, 0)
        halfbody(2 * tt + 1, 1)

    wait_scatter(0)
    wait_scatter(1)
    plsc.subcore_barrier()
    pltpu.sync_copy(
        acc.at[pl.ds(sid * ROWS_PER_SUB, ROWS_PER_SUB)],
        out_hbm.at[pl.ds(cid * NP + sid * ROWS_PER_SUB, ROWS_PER_SUB)])


def _edge_pass(xls, adst, e3, zer, rw, heads, acol, be, nblk):
    mesh = plsc.VectorSubcoreMesh(core_axis_name="c", subcore_axis_name="s")
    cp = pltpu.CompilerParams()
    if "needs_layout_passes" in pltpu.CompilerParams.__dataclass_fields__:
        cp = dataclasses.replace(cp, needs_layout_passes=False)
    if "use_tc_tiling_on_sc" in pltpu.CompilerParams.__dataclass_fields__:
        cp = dataclasses.replace(cp, use_tc_tiling_on_sc=False)
    kern = pl.kernel(
        functools.partial(_edge_body, rw=rw, heads=heads, acol=acol,
                          be=be, nblk=nblk),
        out_type=jax.ShapeDtypeStruct((2 * NP, rw), jnp.float32),
        mesh=mesh,
        scratch_types=[
            pltpu.VMEM((2, be), jnp.int32),
            pltpu.VMEM((2, be), jnp.int32),
            pltpu.VMEM((be,), jnp.int32),
            pltpu.VMEM((be,), jnp.int32),
            pltpu.VMEM((be, rw), jnp.float32),
            pltpu.VMEM((be, rw), jnp.float32),
            pltpu.VMEM((be, 16), jnp.float32),
            pltpu.VMEM((be, 16), jnp.float32),
            pltpu.VMEM((be, rw), jnp.float32),
            pltpu.VMEM((be, rw), jnp.float32),
            pltpu.VMEM((16, be), jnp.float32),
            pltpu.VMEM_SHARED((NP, rw), jnp.float32),
            pltpu.SemaphoreType.DMA,
            pltpu.SemaphoreType.DMA,
            pltpu.SemaphoreType.DMA,
            pltpu.SemaphoreType.DMA,
            pltpu.SemaphoreType.DMA,
            pltpu.SemaphoreType.DMA,
            pltpu.SemaphoreType.DMA,
            pltpu.SemaphoreType.DMA,
        ],
        compiler_params=cp,
    )
    return kern(xls, adst, e3, zer)


def kernel(x, edge_index, W1, att_src1, att_dst1, bias1,
           W2, att_src2, att_dst2, bias2):
    f32 = jnp.float32
    xp = jnp.pad(x, ((0, NP - N), (0, 0)))
    eye8 = jnp.eye(8, dtype=f32)
    asv1 = (att_src1[0][:, :, None] * eye8[:, None, :]).reshape(64, 8)
    adv1 = (att_dst1[0][:, :, None] * eye8[:, None, :]).reshape(64, 8)
    adv1p = jnp.pad(adv1, ((0, 0), (0, 8)))
    r8 = jnp.repeat(eye8, 8, axis=1)
    asv2p = jnp.pad(att_src2[0].reshape(128, 1), ((0, 0), (0, 15)))
    adv2p = jnp.pad(att_dst2[0].reshape(128, 1), ((0, 0), (0, 15)))
    loop = jnp.arange(N, dtype=jnp.int32)
    ei = edge_index.astype(jnp.int32)
    pad = jnp.full((EP - E - N,), DUMMY, jnp.int32)
    srcp = jnp.concatenate([ei[0], loop, pad])
    dstp = jnp.concatenate([ei[1], loop, pad])
    e3a = jnp.stack([srcp.reshape(NTILES * NBLK1, BE1),
                     dstp.reshape(NTILES * NBLK1, BE1)], axis=1)
    e3b = jnp.stack([srcp.reshape(NTILES * NBLK2, BE2),
                     dstp.reshape(NTILES * NBLK2, BE2)], axis=1)
    zer1 = jnp.zeros((NP, RW1), f32)
    zer2 = jnp.zeros((NP, RW2), f32)
    b1 = bias1.reshape(1, 64)
    b2 = bias2.reshape(1, 128)

    xls1, adst1 = _dense1(xp, W1, asv1, adv1p)
    p1 = _edge_pass(xls1, adst1, e3a, zer1, RW1, H1, H1 * C1, BE1, NBLK1)
    xls2, adst2 = _mid(p1[:NP], p1[NP:], b1, W2, r8, asv2p, adv2p)
    p2 = _edge_pass(xls2, adst2, e3b, zer2, RW2, H2, H2 * C2, BE2, NBLK2)
    out = _fin(p2[:NP], p2[NP:], b2)
    return out[:N]
